# Initial kernel scaffold; baseline (speedup 1.0000x reference)
import jax
import jax.numpy as jnp
from jax.experimental import pallas as pl


def kernel(x, edge_index, edge_attr, params):
    raise NotImplementedError("write your pallas kernel here")


# speedup vs baseline: 3.5970x; 3.5970x over previous
import functools

import jax
import jax.numpy as jnp
import numpy as np
from jax import lax
from jax.experimental import pallas as pl
from jax.experimental.pallas import tpu as pltpu
from jax.experimental.pallas import tpu_sc as plsc

NUM_LAYER = 3
EMB = 128
N = 10000
E = 320000
ATOM_TABLE_SIZES = [120, 11, 11, 7, 2, 3]

NTILES = 16
NCORES = 2
NW = NCORES * NTILES
CH = 128
NCH = 79
EPAD = NW * NCH * CH
NP = 10240
ROWS_PER_TILE = NP // NTILES
CW = 16
TTOT = 160
BN = 1280


def _sc_agg_body(h_hbm, src_hbm, dst_hbm, zeros_hbm, out_hbm,
                 src_v, dst_v, buf, acc, sem):
    c = lax.axis_index("c")
    s = lax.axis_index("s")
    wid = c * NTILES + s
    row0 = s * ROWS_PER_TILE
    pltpu.sync_copy(zeros_hbm.at[pl.ds(row0, ROWS_PER_TILE)],
                    acc.at[pl.ds(row0, ROWS_PER_TILE)])
    pltpu.sync_copy(src_hbm.at[wid], src_v)
    pltpu.sync_copy(dst_hbm.at[wid], dst_v)
    plsc.subcore_barrier()

    @pl.loop(0, NCH)
    def _(j):
        pltpu.async_copy(h_hbm.at[src_v.at[j]], buf, sem).wait()
        pltpu.sync_copy(buf, acc.at[dst_v.at[j]], add=True)

    plsc.subcore_barrier()
    pltpu.sync_copy(acc.at[pl.ds(row0, ROWS_PER_TILE)],
                    out_hbm.at[c, pl.ds(row0, ROWS_PER_TILE)])


def _sc_counts_body(tcomb_hbm, ide_hbm, dst_hbm, zeros_hbm, out_hbm,
                    ide_v, dst_v, buf, acc, sem):
    c = lax.axis_index("c")
    s = lax.axis_index("s")
    wid = c * NTILES + s
    row0 = s * ROWS_PER_TILE
    pltpu.sync_copy(zeros_hbm.at[pl.ds(row0, ROWS_PER_TILE)],
                    acc.at[pl.ds(row0, ROWS_PER_TILE)])
    pltpu.sync_copy(ide_hbm.at[wid], ide_v)
    pltpu.sync_copy(dst_hbm.at[wid], dst_v)
    plsc.subcore_barrier()

    @pl.loop(0, NCH)
    def _(j):
        pltpu.async_copy(tcomb_hbm.at[ide_v.at[j]], buf, sem).wait()
        pltpu.sync_copy(buf, acc.at[dst_v.at[j]], add=True)

    plsc.subcore_barrier()
    pltpu.sync_copy(acc.at[pl.ds(row0, ROWS_PER_TILE)],
                    out_hbm.at[c, pl.ds(row0, ROWS_PER_TILE)])


@jax.jit
def _sc_agg(h, src_p, dst_p, zeros128):
    mesh = plsc.VectorSubcoreMesh(core_axis_name="c", subcore_axis_name="s")
    return pl.kernel(
        _sc_agg_body,
        out_type=jax.ShapeDtypeStruct((NCORES, NP, EMB), jnp.float32),
        mesh=mesh,
        scratch_types=[
            pltpu.VMEM((NCH, CH), jnp.int32),
            pltpu.VMEM((NCH, CH), jnp.int32),
            pltpu.VMEM((CH, EMB), jnp.float32),
            pltpu.VMEM_SHARED((NP, EMB), jnp.float32),
            pltpu.SemaphoreType.DMA,
        ],
    )(h, src_p, dst_p, zeros128)


@jax.jit
def _sc_counts(tcomb, ide_p, dst_p, zeros16):
    mesh = plsc.VectorSubcoreMesh(core_axis_name="c", subcore_axis_name="s")
    return pl.kernel(
        _sc_counts_body,
        out_type=jax.ShapeDtypeStruct((NCORES, NP, CW), jnp.float32),
        mesh=mesh,
        scratch_types=[
            pltpu.VMEM((NCH, CH), jnp.int32),
            pltpu.VMEM((NCH, CH), jnp.int32),
            pltpu.VMEM((CH, CW), jnp.float32),
            pltpu.VMEM_SHARED((NP, CW), jnp.float32),
            pltpu.SemaphoreType.DMA,
        ],
        compiler_params=pltpu.CompilerParams(use_tc_tiling_on_sc=False),
    )(tcomb, ide_p, dst_p, zeros16)


def _embed_body(xo_ref, tbl_ref, o_ref):
    io = lax.broadcasted_iota(jnp.int32, (BN, TTOT), 1).astype(jnp.float32)
    oh = jnp.zeros((BN, TTOT), jnp.float32)
    for i in range(6):
        oh = oh + (xo_ref[:, i:i + 1] == io).astype(jnp.float32)
    o_ref[...] = jnp.dot(oh, tbl_ref[...], preferred_element_type=jnp.float32)


@jax.jit
def _embed(xoff_f, tables):
    return pl.pallas_call(
        _embed_body,
        grid=(NP // BN,),
        in_specs=[
            pl.BlockSpec((BN, 6), lambda i: (i, 0)),
            pl.BlockSpec((TTOT, EMB), lambda i: (0, 0)),
        ],
        out_specs=pl.BlockSpec((BN, EMB), lambda i: (i, 0)),
        out_shape=jax.ShapeDtypeStruct((NP, EMB), jnp.float32),
    )(xoff_f, tables)


def _layer_body(p_ref, h_ref, cnt_ref, w1_ref, g1_ref, b1_ref, w2_ref, b2_ref,
                o_ref, *, last):
    sagg = p_ref[0] + p_ref[1] + h_ref[...]
    cnt = cnt_ref[0] + cnt_ref[1]
    z = (jnp.dot(sagg, w1_ref[...], preferred_element_type=jnp.float32)
         + jnp.dot(cnt, g1_ref[...], preferred_element_type=jnp.float32)
         + b1_ref[0:1, :])
    hid = jnp.maximum(z, 0.0)
    o = jnp.dot(hid, w2_ref[...], preferred_element_type=jnp.float32) \
        + b2_ref[0:1, :]
    if not last:
        o = jnp.maximum(o, 0.0)
    o_ref[...] = o


@functools.partial(jax.jit, static_argnames=("last",))
def _layer(p, h, cnts, w1, g1, b1, w2, b2, last):
    return pl.pallas_call(
        functools.partial(_layer_body, last=last),
        grid=(NP // BN,),
        in_specs=[
            pl.BlockSpec((NCORES, BN, EMB), lambda i: (0, i, 0)),
            pl.BlockSpec((BN, EMB), lambda i: (i, 0)),
            pl.BlockSpec((NCORES, BN, CW), lambda i: (0, i, 0)),
            pl.BlockSpec((EMB, 2 * EMB), lambda i: (0, 0)),
            pl.BlockSpec((CW, 2 * EMB), lambda i: (0, 0)),
            pl.BlockSpec((8, 2 * EMB), lambda i: (0, 0)),
            pl.BlockSpec((2 * EMB, EMB), lambda i: (0, 0)),
            pl.BlockSpec((8, EMB), lambda i: (0, 0)),
        ],
        out_specs=pl.BlockSpec((BN, EMB), lambda i: (i, 0)),
        out_shape=jax.ShapeDtypeStruct((NP, EMB), jnp.float32),
    )(p, h, cnts, w1, g1, b1, w2, b2)


_TCOMB = np.zeros((18, CW), np.float32)
for _t in range(6):
    for _d in range(3):
        _TCOMB[_t * 3 + _d, _t] = 1.0
        _TCOMB[_t * 3 + _d, 8 + _d] = 1.0

_OFFS = np.cumsum([0] + ATOM_TABLE_SIZES[:-1]).astype(np.int32)


def kernel(x, edge_index, edge_attr, params):
    f32 = jnp.float32
    i32 = jnp.int32

    xoff = x.astype(i32) + jnp.asarray(_OFFS)[None, :]
    xoff_f = jnp.zeros((NP, 6), f32).at[:N].set(xoff.astype(f32))

    src = edge_index[0].astype(i32)
    dst = edge_index[1].astype(i32)
    ide = edge_attr[:, 0].astype(i32) * 3 + edge_attr[:, 1].astype(i32)
    pad = EPAD - E
    src_p = jnp.concatenate([src, jnp.zeros((pad,), i32)]).reshape(NW, NCH, CH)
    dst_p = jnp.concatenate([dst, jnp.full((pad,), N, i32)]).reshape(NW, NCH, CH)
    ide_p = jnp.concatenate([ide, jnp.zeros((pad,), i32)]).reshape(NW, NCH, CH)

    zeros128 = jnp.zeros((NP, EMB), f32)
    zeros16 = jnp.zeros((NP, CW), f32)
    tcomb = jnp.asarray(_TCOMB)

    tables = jnp.zeros((TTOT, EMB), f32)
    for i, t in enumerate(params['atom_embs']):
        tables = tables.at[int(_OFFS[i]):int(_OFFS[i]) + t.shape[0]].set(t)

    scale = 1.0 / jnp.sqrt(1.0 + 1e-5)
    lw = []
    for l in range(NUM_LAYER):
        lp = params['layers'][l]
        e1, e2 = lp['edge_emb1'], lp['edge_emb2']
        e16 = jnp.zeros((CW, EMB), f32).at[0:6].set(e1).at[8:11].set(e2)
        g1 = jnp.dot(e16, lp['W1'])
        const = e1[4] + e2[0]
        b1p = lp['b1'] + jnp.dot(const, lp['W1'])
        gs = lp['gamma'] * scale
        w2p = lp['W2'] * gs[None, :]
        b2p = lp['b2'] * gs + lp['beta']
        lw.append((lp['W1'], g1,
                   jnp.broadcast_to(b1p[None, :], (8, 2 * EMB)),
                   w2p,
                   jnp.broadcast_to(b2p[None, :], (8, EMB))))

    h = _embed(xoff_f, tables)
    cnts = _sc_counts(tcomb, ide_p, dst_p, zeros16)
    for l in range(NUM_LAYER):
        p = _sc_agg(h, src_p, dst_p, zeros128)
        w1, g1, b1p, w2p, b2p = lw[l]
        h = _layer(p, h, cnts, w1, g1, b1p, w2p, b2p, l == NUM_LAYER - 1)
    return h[:N]


# speedup vs baseline: 3.7865x; 1.0527x over previous
import functools

import jax
import jax.numpy as jnp
import numpy as np
from jax import lax
from jax.experimental import pallas as pl
from jax.experimental.pallas import tpu as pltpu
from jax.experimental.pallas import tpu_sc as plsc

NUM_LAYER = 3
EMB = 128
N = 10000
E = 320000
ATOM_TABLE_SIZES = [120, 11, 11, 7, 2, 3]

NTILES = 16
NCORES = 2
NW = NCORES * NTILES
CH = 120
NSLOT = 3
NCH = 84
EPAD = NW * NCH * CH
NP = 10112
ROWS_PER_TILE = NP // NTILES
TTOT = 160
BN = 1264


def _sc_agg_body(h_hbm, idx_hbm, zeros_hbm, out_hbm,
                 i0, i1, i2, b0, b1, b2,
                 is0, is1, is2, gs0, gs1, gs2, ss0, ss1, ss2, acc):
    ibufs = (i0, i1, i2)
    isems = (is0, is1, is2)
    bufs = (b0, b1, b2)
    gsems = (gs0, gs1, gs2)
    ssems = (ss0, ss1, ss2)
    c = lax.axis_index("c")
    s = lax.axis_index("s")
    wid = c * NTILES + s
    row0 = s * ROWS_PER_TILE

    def wait_idx(sl):
        pltpu.make_async_copy(idx_hbm.at[wid, 0], ibufs[sl], isems[sl]).wait()

    def wait_gather(sl):
        pltpu.make_async_copy(h_hbm.at[ibufs[sl].at[0]], bufs[sl],
                              gsems[sl]).wait()

    def wait_scatter(sl):
        pltpu.make_async_copy(bufs[sl], acc.at[ibufs[sl].at[1]],
                              ssems[sl]).wait()

    for sl in range(NSLOT):
        pltpu.async_copy(idx_hbm.at[wid, sl], ibufs[sl], isems[sl])
    pltpu.sync_copy(zeros_hbm.at[pl.ds(row0, ROWS_PER_TILE)],
                    acc.at[pl.ds(row0, ROWS_PER_TILE)])
    plsc.subcore_barrier()

    for sl in range(NSLOT):
        wait_idx(sl)
        pltpu.async_copy(h_hbm.at[ibufs[sl].at[0]], bufs[sl], gsems[sl])
    for sl in range(NSLOT):
        wait_gather(sl)
        pltpu.async_copy(bufs[sl], acc.at[ibufs[sl].at[1]], ssems[sl],
                         add=True)

    @pl.loop(NSLOT, NCH, step=NSLOT)
    def _(j):
        for sl in range(NSLOT):
            wait_scatter(sl)
            pltpu.async_copy(idx_hbm.at[wid, j + sl], ibufs[sl], isems[sl])
        for sl in range(NSLOT):
            wait_idx(sl)
            pltpu.async_copy(h_hbm.at[ibufs[sl].at[0]], bufs[sl], gsems[sl])
        for sl in range(NSLOT):
            wait_gather(sl)
            pltpu.async_copy(bufs[sl], acc.at[ibufs[sl].at[1]], ssems[sl],
                             add=True)

    for sl in range(NSLOT):
        wait_scatter(sl)

    plsc.subcore_barrier()
    pltpu.sync_copy(acc.at[pl.ds(row0, ROWS_PER_TILE)],
                    out_hbm.at[c, pl.ds(row0, ROWS_PER_TILE)])


@jax.jit
def _sc_agg(table, idx_p, zeros128):
    mesh = plsc.VectorSubcoreMesh(core_axis_name="c", subcore_axis_name="s")
    return pl.kernel(
        _sc_agg_body,
        out_type=jax.ShapeDtypeStruct((NCORES, NP, EMB), jnp.float32),
        mesh=mesh,
        scratch_types=(
            [pltpu.VMEM((2, CH), jnp.int32)] * NSLOT
            + [pltpu.VMEM((CH, EMB), jnp.float32)] * NSLOT
            + [pltpu.SemaphoreType.DMA] * (3 * NSLOT)
            + [pltpu.VMEM_SHARED((NP, EMB), jnp.float32)]
        ),
    )(table, idx_p, zeros128)


def _embed_body(xo_ref, tbl_ref, o_ref):
    io = lax.broadcasted_iota(jnp.int32, (BN, TTOT), 1).astype(jnp.float32)
    oh = jnp.zeros((BN, TTOT), jnp.float32)
    for i in range(6):
        oh = oh + (xo_ref[:, i:i + 1] == io).astype(jnp.float32)
    o_ref[...] = jnp.dot(oh, tbl_ref[...], preferred_element_type=jnp.float32)


@jax.jit
def _embed(xoff_f, tables):
    return pl.pallas_call(
        _embed_body,
        grid=(NP // BN,),
        in_specs=[
            pl.BlockSpec((BN, 6), lambda i: (i, 0)),
            pl.BlockSpec((TTOT, EMB), lambda i: (0, 0)),
        ],
        out_specs=pl.BlockSpec((BN, EMB), lambda i: (i, 0)),
        out_shape=jax.ShapeDtypeStruct((NP, EMB), jnp.float32),
    )(xoff_f, tables)


def _layer_body(p_ref, h_ref, cnt_ref, w1_ref, g1_ref, b1_ref, w2_ref, b2_ref,
                o_ref, *, last):
    sagg = p_ref[0] + p_ref[1] + h_ref[...]
    cnt = cnt_ref[0] + cnt_ref[1]
    z = (jnp.dot(sagg, w1_ref[...], preferred_element_type=jnp.float32)
         + jnp.dot(cnt, g1_ref[...], preferred_element_type=jnp.float32)
         + b1_ref[0:1, :])
    hid = jnp.maximum(z, 0.0)
    o = jnp.dot(hid, w2_ref[...], preferred_element_type=jnp.float32) \
        + b2_ref[0:1, :]
    if not last:
        o = jnp.maximum(o, 0.0)
    o_ref[...] = o


@functools.partial(jax.jit, static_argnames=("last",))
def _layer(p, h, cnts, w1, g1, b1, w2, b2, last):
    return pl.pallas_call(
        functools.partial(_layer_body, last=last),
        grid=(NP // BN,),
        in_specs=[
            pl.BlockSpec((NCORES, BN, EMB), lambda i: (0, i, 0)),
            pl.BlockSpec((BN, EMB), lambda i: (i, 0)),
            pl.BlockSpec((NCORES, BN, EMB), lambda i: (0, i, 0)),
            pl.BlockSpec((EMB, 2 * EMB), lambda i: (0, 0)),
            pl.BlockSpec((EMB, 2 * EMB), lambda i: (0, 0)),
            pl.BlockSpec((8, 2 * EMB), lambda i: (0, 0)),
            pl.BlockSpec((2 * EMB, EMB), lambda i: (0, 0)),
            pl.BlockSpec((8, EMB), lambda i: (0, 0)),
        ],
        out_specs=pl.BlockSpec((BN, EMB), lambda i: (i, 0)),
        out_shape=jax.ShapeDtypeStruct((NP, EMB), jnp.float32),
    )(p, h, cnts, w1, g1, b1, w2, b2)


_TCOMB = np.zeros((18, EMB), np.float32)
for _t in range(6):
    for _d in range(3):
        _TCOMB[_t * 3 + _d, _t] = 1.0
        _TCOMB[_t * 3 + _d, 8 + _d] = 1.0

_OFFS = np.cumsum([0] + ATOM_TABLE_SIZES[:-1]).astype(np.int32)


def kernel(x, edge_index, edge_attr, params):
    f32 = jnp.float32
    i32 = jnp.int32

    xoff = x.astype(i32) + jnp.asarray(_OFFS)[None, :]
    xoff_f = jnp.zeros((NP, 6), f32).at[:N].set(xoff.astype(f32))

    src = edge_index[0].astype(i32)
    dst = edge_index[1].astype(i32)
    ide = edge_attr[:, 0].astype(i32) * 3 + edge_attr[:, 1].astype(i32)
    pad = EPAD - E
    src_p = jnp.concatenate([src, jnp.zeros((pad,), i32)]).reshape(NW, NCH, CH)
    dst_p = jnp.concatenate([dst, jnp.full((pad,), N, i32)]).reshape(NW, NCH, CH)
    ide_p = jnp.concatenate([ide, jnp.zeros((pad,), i32)]).reshape(NW, NCH, CH)
    dummy = jnp.concatenate([
        jnp.zeros((NW, NSLOT, 1, CH), i32),
        jnp.full((NW, NSLOT, 1, CH), N, i32)], axis=2)
    sd_p = jnp.concatenate(
        [jnp.stack([src_p, dst_p], axis=2), dummy], axis=1)
    ic_p = jnp.concatenate(
        [jnp.stack([ide_p, dst_p], axis=2), dummy], axis=1)

    zeros128 = jnp.zeros((NP, EMB), f32)
    tcomb = jnp.zeros((NP, EMB), f32).at[:18].set(jnp.asarray(_TCOMB))

    tables = jnp.zeros((TTOT, EMB), f32)
    for i, t in enumerate(params['atom_embs']):
        tables = tables.at[int(_OFFS[i]):int(_OFFS[i]) + t.shape[0]].set(t)

    scale = 1.0 / jnp.sqrt(1.0 + 1e-5)
    lw = []
    for l in range(NUM_LAYER):
        lp = params['layers'][l]
        e1, e2 = lp['edge_emb1'], lp['edge_emb2']
        e128 = jnp.zeros((EMB, EMB), f32).at[0:6].set(e1).at[8:11].set(e2)
        g1 = jnp.dot(e128, lp['W1'])
        const = e1[4] + e2[0]
        b1p = lp['b1'] + jnp.dot(const, lp['W1'])
        gs = lp['gamma'] * scale
        w2p = lp['W2'] * gs[None, :]
        b2p = lp['b2'] * gs + lp['beta']
        lw.append((lp['W1'], g1,
                   jnp.broadcast_to(b1p[None, :], (8, 2 * EMB)),
                   w2p,
                   jnp.broadcast_to(b2p[None, :], (8, EMB))))

    h = _embed(xoff_f, tables)
    cnts = _sc_agg(tcomb, ic_p, zeros128)
    h, cnts = lax.optimization_barrier((h, cnts))
    for l in range(NUM_LAYER):
        p = _sc_agg(h, sd_p, zeros128)
        w1, g1, b1p, w2p, b2p = lw[l]
        h = _layer(p, h, cnts, w1, g1, b1p, w2p, b2p, l == NUM_LAYER - 1)
    return h[:N]
